# Initial kernel scaffold; baseline (speedup 1.0000x reference)
import jax
import jax.numpy as jnp
from jax.experimental import pallas as pl


def kernel(audio_semantic, eeg_semantic, embedding):
    raise NotImplementedError("write your pallas kernel here")


# speedup vs baseline: 2.0829x; 2.0829x over previous
import functools

import jax
import jax.numpy as jnp
from jax.experimental import pallas as pl

COMMITMENT_COST = 0.25
EPSILON = 1e-05


def _vq_main(z_ref, emb_ref, e2_ref, ph_ref, cnt_ref, idx_ref, q_ref, *, T, M):
    r = pl.program_id(1)
    x = z_ref[0]
    emb = emb_ref[...]
    RB = x.shape[0]

    x2 = jnp.sum(x * x, axis=1, keepdims=True)
    xz = jax.lax.dot_general(x, emb, (((1,), (1,)), ((), ())),
                             preferred_element_type=jnp.float32)
    dist = e2_ref[...] + x2 - 2.0 * xz

    dmin = jnp.min(dist, axis=1, keepdims=True)
    nd = -jnp.sqrt(dist)
    ee = jnp.exp(nd + jnp.sqrt(dmin))
    s = jnp.sum(ee, axis=1, keepdims=True)
    php = jnp.sum(ee * (1.0 / s), axis=0, keepdims=True) * (1.0 / T)

    iota = jax.lax.broadcasted_iota(jnp.int32, (RB, M), 1)
    idxm = jnp.where(dist == dmin, iota, M)
    idx = jnp.min(idxm, axis=1, keepdims=True)
    onehot = (iota == idx).astype(jnp.float32)
    cnt = jnp.sum(onehot, axis=0, keepdims=True)
    q = jax.lax.dot_general(onehot, emb, (((1,), (0,)), ((), ())),
                            preferred_element_type=jnp.float32)

    idx_ref[0] = idx
    q_ref[0] = q

    @pl.when(r == 0)
    def _():
        ph_ref[...] = jnp.zeros_like(ph_ref)
        cnt_ref[...] = jnp.zeros_like(cnt_ref)

    ph_ref[0] += php
    cnt_ref[0] += cnt


def kernel(audio_semantic, eeg_semantic, embedding):
    B, T, D = audio_semantic.shape
    M = embedding.shape[0]
    BT = B * T
    RB = 192
    nr = T // RB
    G = 2 * B

    z = jnp.concatenate([audio_semantic, eeg_semantic], axis=0)
    e2 = jnp.sum(embedding * embedding, axis=1)[None, :]

    ph, cnt, idx, q = pl.pallas_call(
        functools.partial(_vq_main, T=T, M=M),
        grid=(G, nr),
        in_specs=[
            pl.BlockSpec((1, RB, D), lambda g, r: (g, r, 0)),
            pl.BlockSpec((M, D), lambda g, r: (0, 0)),
            pl.BlockSpec((1, M), lambda g, r: (0, 0)),
        ],
        out_specs=[
            pl.BlockSpec((1, 1, M), lambda g, r: (g, 0, 0)),
            pl.BlockSpec((1, 1, M), lambda g, r: (g, 0, 0)),
            pl.BlockSpec((1, RB, 1), lambda g, r: (g, r, 0)),
            pl.BlockSpec((1, RB, D), lambda g, r: (g, r, 0)),
        ],
        out_shape=[
            jax.ShapeDtypeStruct((G, 1, M), jnp.float32),
            jax.ShapeDtypeStruct((G, 1, M), jnp.float32),
            jax.ShapeDtypeStruct((G, T, 1), jnp.int32),
            jax.ShapeDtypeStruct((G, T, D), jnp.float32),
        ],
    )(z, embedding, e2)

    a_ph, e_ph = ph[:B, 0], ph[B:, 0]
    a_counts, e_counts = cnt[:B, 0], cnt[B:, 0]
    a_q, e_q = q[:B], q[B:]

    Scode = a_ph @ jnp.log(e_ph.T + 1e-10) + e_ph @ jnp.log(a_ph.T + 1e-10)
    MaxScode = jnp.max(-Scode)
    EScode = jnp.exp(Scode + MaxScode)
    EScode_dim1sum = jnp.sum(EScode, axis=1)
    Lcmcm = -jnp.sum(jnp.log(jnp.diagonal(EScode) / (EScode_dim1sum + EPSILON))) / B
    cmcm_loss = 0.5 * Lcmcm

    a_mode = jnp.argmax(a_counts, axis=1)
    e_mode = jnp.argmax(e_counts, axis=1)
    equal_num = jnp.sum(a_mode == e_mode)

    def mse(x, y):
        return jnp.mean((x - y) ** 2)

    a_e_latent_loss = mse(audio_semantic, a_q)
    ae_e_latent_loss = mse(audio_semantic, e_q)
    a_loss = COMMITMENT_COST * (2.0 * a_e_latent_loss + ae_e_latent_loss)
    e_e_latent_loss = mse(eeg_semantic, e_q)
    ea_e_latent_loss = mse(eeg_semantic, a_q)
    e_loss = COMMITMENT_COST * (2.0 * e_e_latent_loss + ea_e_latent_loss)

    a_quantized_st = audio_semantic + (a_q - audio_semantic)
    e_quantized_st = eeg_semantic + (e_q - eeg_semantic)

    a_avg_probs = jnp.sum(a_counts, axis=0) / BT
    a_perplexity = jnp.exp(-jnp.sum(a_avg_probs * jnp.log(a_avg_probs + 1e-10)))
    e_avg_probs = jnp.sum(e_counts, axis=0) / BT
    e_perplexity = jnp.exp(-jnp.sum(e_avg_probs * jnp.log(e_avg_probs + 1e-10)))

    return (a_quantized_st, e_quantized_st, a_loss, e_loss,
            a_perplexity, e_perplexity, cmcm_loss, equal_num)


# speedup vs baseline: 2.1105x; 1.0132x over previous
import functools

import jax
import jax.numpy as jnp
from jax.experimental import pallas as pl
from jax.experimental.pallas import tpu as pltpu
from jax.experimental.pallas import tpu_sc as plsc

COMMITMENT_COST = 0.25
EPSILON = 1e-05


def _vq_main(z_ref, emb_ref, e2_ref, ph_ref, cnt_ref, idx_ref, *, T, M):
    r = pl.program_id(1)
    x = z_ref[0]
    emb = emb_ref[...]
    RB = x.shape[0]

    x2 = jnp.sum(x * x, axis=1, keepdims=True)
    xz = jax.lax.dot_general(x, emb, (((1,), (1,)), ((), ())),
                             preferred_element_type=jnp.float32)
    dist = e2_ref[...] + x2 - 2.0 * xz

    dmin = jnp.min(dist, axis=1, keepdims=True)
    nd = -jnp.sqrt(dist)
    ee = jnp.exp(nd + jnp.sqrt(dmin))
    s = jnp.sum(ee, axis=1, keepdims=True)
    php = jnp.sum(ee * (1.0 / s), axis=0, keepdims=True) * (1.0 / T)

    iota = jax.lax.broadcasted_iota(jnp.int32, (RB, M), 1)
    idxm = jnp.where(dist == dmin, iota, M)
    idx = jnp.min(idxm, axis=1, keepdims=True)
    onehot = (iota == idx).astype(jnp.float32)
    cnt = jnp.sum(onehot, axis=0, keepdims=True)

    idx_ref[0] = idx

    @pl.when(r == 0)
    def _():
        ph_ref[...] = jnp.zeros_like(ph_ref)
        cnt_ref[...] = jnp.zeros_like(cnt_ref)

    ph_ref[0] += php
    cnt_ref[0] += cnt


def _sc_gather(embedding, idx_flat, n, d):
    W = 128
    mesh = plsc.VectorSubcoreMesh(core_axis_name="c", subcore_axis_name="s")

    @functools.partial(
        pl.kernel,
        out_type=jax.ShapeDtypeStruct((n, d), jnp.float32),
        mesh=mesh,
    )
    def _gather_kernel(emb_hbm, i_hbm, o_hbm):
        def body(i_vmem, o_vmem):
            pltpu.sync_copy(emb_hbm.at[i_vmem.at[0]], o_vmem)

        pltpu.emit_pipeline(
            body,
            grid=(n // W,),
            in_specs=[pl.BlockSpec((1, W), lambda i: (0, i))],
            out_specs=[pl.BlockSpec((W, d), lambda i: (i, 0))],
            core_axis_name=("c", "s"),
            dimension_semantics=(pltpu.PARALLEL,),
        )(i_hbm, o_hbm)

    return _gather_kernel(embedding, idx_flat.reshape(1, n))


def kernel(audio_semantic, eeg_semantic, embedding):
    B, T, D = audio_semantic.shape
    M = embedding.shape[0]
    BT = B * T
    RB = 192
    nr = T // RB
    G = 2 * B

    z = jnp.concatenate([audio_semantic, eeg_semantic], axis=0)
    e2 = jnp.sum(embedding * embedding, axis=1)[None, :]

    ph, cnt, idx = pl.pallas_call(
        functools.partial(_vq_main, T=T, M=M),
        grid=(G, nr),
        in_specs=[
            pl.BlockSpec((1, RB, D), lambda g, r: (g, r, 0)),
            pl.BlockSpec((M, D), lambda g, r: (0, 0)),
            pl.BlockSpec((1, M), lambda g, r: (0, 0)),
        ],
        out_specs=[
            pl.BlockSpec((1, 1, M), lambda g, r: (g, 0, 0)),
            pl.BlockSpec((1, 1, M), lambda g, r: (g, 0, 0)),
            pl.BlockSpec((1, RB, 1), lambda g, r: (g, r, 0)),
        ],
        out_shape=[
            jax.ShapeDtypeStruct((G, 1, M), jnp.float32),
            jax.ShapeDtypeStruct((G, 1, M), jnp.float32),
            jax.ShapeDtypeStruct((G, T, 1), jnp.int32),
        ],
    )(z, embedding, e2)

    a_ph, e_ph = ph[:B, 0], ph[B:, 0]
    a_counts, e_counts = cnt[:B, 0], cnt[B:, 0]
    q = _sc_gather(embedding, idx, G * T, D).reshape(G, T, D)
    a_q, e_q = q[:B], q[B:]

    Scode = a_ph @ jnp.log(e_ph.T + 1e-10) + e_ph @ jnp.log(a_ph.T + 1e-10)
    MaxScode = jnp.max(-Scode)
    EScode = jnp.exp(Scode + MaxScode)
    EScode_dim1sum = jnp.sum(EScode, axis=1)
    Lcmcm = -jnp.sum(jnp.log(jnp.diagonal(EScode) / (EScode_dim1sum + EPSILON))) / B
    cmcm_loss = 0.5 * Lcmcm

    a_mode = jnp.argmax(a_counts, axis=1)
    e_mode = jnp.argmax(e_counts, axis=1)
    equal_num = jnp.sum(a_mode == e_mode)

    def mse(x, y):
        return jnp.mean((x - y) ** 2)

    a_e_latent_loss = mse(audio_semantic, a_q)
    ae_e_latent_loss = mse(audio_semantic, e_q)
    a_loss = COMMITMENT_COST * (2.0 * a_e_latent_loss + ae_e_latent_loss)
    e_e_latent_loss = mse(eeg_semantic, e_q)
    ea_e_latent_loss = mse(eeg_semantic, a_q)
    e_loss = COMMITMENT_COST * (2.0 * e_e_latent_loss + ea_e_latent_loss)

    a_quantized_st = audio_semantic + (a_q - audio_semantic)
    e_quantized_st = eeg_semantic + (e_q - eeg_semantic)

    a_avg_probs = jnp.sum(a_counts, axis=0) / BT
    a_perplexity = jnp.exp(-jnp.sum(a_avg_probs * jnp.log(a_avg_probs + 1e-10)))
    e_avg_probs = jnp.sum(e_counts, axis=0) / BT
    e_perplexity = jnp.exp(-jnp.sum(e_avg_probs * jnp.log(e_avg_probs + 1e-10)))

    return (a_quantized_st, e_quantized_st, a_loss, e_loss,
            a_perplexity, e_perplexity, cmcm_loss, equal_num)


# speedup vs baseline: 2.3177x; 1.0982x over previous
import functools

import jax
import jax.numpy as jnp
from jax.experimental import pallas as pl
from jax.experimental.pallas import tpu as pltpu
from jax.experimental.pallas import tpu_sc as plsc

COMMITMENT_COST = 0.25
EPSILON = 1e-05


def _vq_main(z_ref, emb_ref, e2_ref, ph_ref, cnt_ref, idx_ref, *, T, M):
    r = pl.program_id(1)
    x = z_ref[0]
    emb = emb_ref[...]
    RB = x.shape[0]

    x2 = jnp.sum(x * x, axis=1, keepdims=True)
    xz = jax.lax.dot_general(x, emb, (((1,), (1,)), ((), ())),
                             preferred_element_type=jnp.float32)
    dist = e2_ref[...] + x2 - 2.0 * xz

    dmin = jnp.min(dist, axis=1, keepdims=True)
    ee = jnp.exp(jnp.sqrt(dmin) - jnp.sqrt(dist))
    s = jnp.sum(ee, axis=1, keepdims=True)
    recip = jnp.transpose((1.0 / T) / s)
    php = jax.lax.dot_general(recip, ee, (((1,), (0,)), ((), ())),
                              preferred_element_type=jnp.float32)

    iota = jax.lax.broadcasted_iota(jnp.int32, (RB, M), 1)
    idxm = jnp.where(dist == dmin, iota, M)
    idx = jnp.min(idxm, axis=1, keepdims=True)
    onehot = (iota == idx).astype(jnp.float32)
    ones_row = jnp.ones((1, RB), jnp.float32)
    cnt = jax.lax.dot_general(ones_row, onehot, (((1,), (0,)), ((), ())),
                              preferred_element_type=jnp.float32)

    idx_ref[0] = idx

    @pl.when(r == 0)
    def _():
        ph_ref[...] = jnp.zeros_like(ph_ref)
        cnt_ref[...] = jnp.zeros_like(cnt_ref)

    ph_ref[0] += php
    cnt_ref[0] += cnt


def _sc_gather(embedding, idx_flat, n, d):
    W = 128
    mesh = plsc.VectorSubcoreMesh(core_axis_name="c", subcore_axis_name="s")

    @functools.partial(
        pl.kernel,
        out_type=jax.ShapeDtypeStruct((n, d), jnp.float32),
        mesh=mesh,
    )
    def _gather_kernel(emb_hbm, i_hbm, o_hbm):
        def body(i_vmem, o_vmem):
            pltpu.sync_copy(emb_hbm.at[i_vmem.at[0]], o_vmem)

        pltpu.emit_pipeline(
            body,
            grid=(n // W,),
            in_specs=[pl.BlockSpec((1, W), lambda i: (0, i))],
            out_specs=[pl.BlockSpec((W, d), lambda i: (i, 0))],
            core_axis_name=("c", "s"),
            dimension_semantics=(pltpu.PARALLEL,),
        )(i_hbm, o_hbm)

    return _gather_kernel(embedding, idx_flat.reshape(1, n))


def kernel(audio_semantic, eeg_semantic, embedding):
    B, T, D = audio_semantic.shape
    M = embedding.shape[0]
    BT = B * T
    RB = 192
    nr = T // RB
    G = 2 * B

    z = jnp.concatenate([audio_semantic, eeg_semantic], axis=0)
    e2 = jnp.sum(embedding * embedding, axis=1)[None, :]

    ph, cnt, idx = pl.pallas_call(
        functools.partial(_vq_main, T=T, M=M),
        grid=(G, nr),
        in_specs=[
            pl.BlockSpec((1, RB, D), lambda g, r: (g, r, 0)),
            pl.BlockSpec((M, D), lambda g, r: (0, 0)),
            pl.BlockSpec((1, M), lambda g, r: (0, 0)),
        ],
        out_specs=[
            pl.BlockSpec((1, 1, M), lambda g, r: (g, 0, 0)),
            pl.BlockSpec((1, 1, M), lambda g, r: (g, 0, 0)),
            pl.BlockSpec((1, RB, 1), lambda g, r: (g, r, 0)),
        ],
        out_shape=[
            jax.ShapeDtypeStruct((G, 1, M), jnp.float32),
            jax.ShapeDtypeStruct((G, 1, M), jnp.float32),
            jax.ShapeDtypeStruct((G, T, 1), jnp.int32),
        ],
    )(z, embedding, e2)

    a_ph, e_ph = ph[:B, 0], ph[B:, 0]
    a_counts, e_counts = cnt[:B, 0], cnt[B:, 0]
    q = _sc_gather(embedding, idx, G * T, D).reshape(G, T, D)
    a_q, e_q = q[:B], q[B:]

    Scode = a_ph @ jnp.log(e_ph.T + 1e-10) + e_ph @ jnp.log(a_ph.T + 1e-10)
    MaxScode = jnp.max(-Scode)
    EScode = jnp.exp(Scode + MaxScode)
    EScode_dim1sum = jnp.sum(EScode, axis=1)
    Lcmcm = -jnp.sum(jnp.log(jnp.diagonal(EScode) / (EScode_dim1sum + EPSILON))) / B
    cmcm_loss = 0.5 * Lcmcm

    a_mode = jnp.argmax(a_counts, axis=1)
    e_mode = jnp.argmax(e_counts, axis=1)
    equal_num = jnp.sum(a_mode == e_mode)

    def mse(x, y):
        return jnp.mean((x - y) ** 2)

    a_e_latent_loss = mse(audio_semantic, a_q)
    ae_e_latent_loss = mse(audio_semantic, e_q)
    a_loss = COMMITMENT_COST * (2.0 * a_e_latent_loss + ae_e_latent_loss)
    e_e_latent_loss = mse(eeg_semantic, e_q)
    ea_e_latent_loss = mse(eeg_semantic, a_q)
    e_loss = COMMITMENT_COST * (2.0 * e_e_latent_loss + ea_e_latent_loss)

    a_quantized_st = audio_semantic + (a_q - audio_semantic)
    e_quantized_st = eeg_semantic + (e_q - eeg_semantic)

    a_avg_probs = jnp.sum(a_counts, axis=0) / BT
    a_perplexity = jnp.exp(-jnp.sum(a_avg_probs * jnp.log(a_avg_probs + 1e-10)))
    e_avg_probs = jnp.sum(e_counts, axis=0) / BT
    e_perplexity = jnp.exp(-jnp.sum(e_avg_probs * jnp.log(e_avg_probs + 1e-10)))

    return (a_quantized_st, e_quantized_st, a_loss, e_loss,
            a_perplexity, e_perplexity, cmcm_loss, equal_num)


# speedup vs baseline: 2.4470x; 1.0558x over previous
import functools

import jax
import jax.numpy as jnp
from jax.experimental import pallas as pl
from jax.experimental.pallas import tpu as pltpu
from jax.experimental.pallas import tpu_sc as plsc

COMMITMENT_COST = 0.25
EPSILON = 1e-05


def _vq_main(z_ref, emb_ref, e2_ref, ph_ref, cnt_ref, idx_ref, *, T, M):
    r = pl.program_id(1)
    x = z_ref[0]
    emb = emb_ref[...]
    RB = x.shape[0]

    SCALE = 1.4426950408889634 ** 2
    x2 = jnp.sum(x * x, axis=1, keepdims=True) * SCALE
    xz = jax.lax.dot_general(x, emb, (((1,), (1,)), ((), ())),
                             preferred_element_type=jnp.float32)
    dist = e2_ref[...] + x2 - (2.0 * SCALE) * xz

    dmin = jnp.min(dist, axis=1, keepdims=True)
    ee = jnp.exp2(jnp.sqrt(dmin) - jnp.sqrt(dist))
    s = jnp.sum(ee, axis=1, keepdims=True)
    recip = jnp.transpose((1.0 / T) / s)
    php = jax.lax.dot_general(recip, ee, (((1,), (0,)), ((), ())),
                              preferred_element_type=jnp.float32)

    iota = jax.lax.broadcasted_iota(jnp.int32, (RB, M), 1)
    idxm = jnp.where(dist == dmin, iota, M)
    idx = jnp.min(idxm, axis=1, keepdims=True)
    onehot = (iota == idx).astype(jnp.float32)
    ones_row = jnp.ones((1, RB), jnp.float32)
    cnt = jax.lax.dot_general(ones_row, onehot, (((1,), (0,)), ((), ())),
                              preferred_element_type=jnp.float32)

    idx_ref[0] = idx

    @pl.when(r == 0)
    def _():
        ph_ref[...] = jnp.zeros_like(ph_ref)
        cnt_ref[...] = jnp.zeros_like(cnt_ref)

    ph_ref[0] += php
    cnt_ref[0] += cnt


def _sc_gather(embedding, idx_flat, n, d):
    W = 128
    mesh = plsc.VectorSubcoreMesh(core_axis_name="c", subcore_axis_name="s")

    @functools.partial(
        pl.kernel,
        out_type=jax.ShapeDtypeStruct((n, d), jnp.float32),
        mesh=mesh,
    )
    def _gather_kernel(emb_hbm, i_hbm, o_hbm):
        def body(i_vmem, o_vmem):
            pltpu.sync_copy(emb_hbm.at[i_vmem.at[0]], o_vmem)

        pltpu.emit_pipeline(
            body,
            grid=(n // W,),
            in_specs=[pl.BlockSpec((1, W), lambda i: (0, i))],
            out_specs=[pl.BlockSpec((W, d), lambda i: (i, 0))],
            core_axis_name=("c", "s"),
            dimension_semantics=(pltpu.PARALLEL,),
        )(i_hbm, o_hbm)

    return _gather_kernel(embedding, idx_flat.reshape(1, n))


def kernel(audio_semantic, eeg_semantic, embedding):
    B, T, D = audio_semantic.shape
    M = embedding.shape[0]
    BT = B * T
    RB = 288
    nr = T // RB
    G = 2 * B

    z = jnp.concatenate([audio_semantic, eeg_semantic], axis=0)
    e2 = (jnp.sum(embedding * embedding, axis=1) * 1.4426950408889634 ** 2)[None, :]

    ph, cnt, idx = pl.pallas_call(
        functools.partial(_vq_main, T=T, M=M),
        grid=(G, nr),
        in_specs=[
            pl.BlockSpec((1, RB, D), lambda g, r: (g, r, 0)),
            pl.BlockSpec((M, D), lambda g, r: (0, 0)),
            pl.BlockSpec((1, M), lambda g, r: (0, 0)),
        ],
        out_specs=[
            pl.BlockSpec((1, 1, M), lambda g, r: (g, 0, 0)),
            pl.BlockSpec((1, 1, M), lambda g, r: (g, 0, 0)),
            pl.BlockSpec((1, RB, 1), lambda g, r: (g, r, 0)),
        ],
        out_shape=[
            jax.ShapeDtypeStruct((G, 1, M), jnp.float32),
            jax.ShapeDtypeStruct((G, 1, M), jnp.float32),
            jax.ShapeDtypeStruct((G, T, 1), jnp.int32),
        ],
    )(z, embedding, e2)

    a_ph, e_ph = ph[:B, 0], ph[B:, 0]
    a_counts, e_counts = cnt[:B, 0], cnt[B:, 0]
    q = _sc_gather(embedding, idx, G * T, D).reshape(G, T, D)
    a_q, e_q = q[:B], q[B:]

    Scode = a_ph @ jnp.log(e_ph.T + 1e-10) + e_ph @ jnp.log(a_ph.T + 1e-10)
    MaxScode = jnp.max(-Scode)
    EScode = jnp.exp(Scode + MaxScode)
    EScode_dim1sum = jnp.sum(EScode, axis=1)
    Lcmcm = -jnp.sum(jnp.log(jnp.diagonal(EScode) / (EScode_dim1sum + EPSILON))) / B
    cmcm_loss = 0.5 * Lcmcm

    a_mode = jnp.argmax(a_counts, axis=1)
    e_mode = jnp.argmax(e_counts, axis=1)
    equal_num = jnp.sum(a_mode == e_mode)

    def mse(x, y):
        return jnp.mean((x - y) ** 2)

    a_e_latent_loss = mse(audio_semantic, a_q)
    ae_e_latent_loss = mse(audio_semantic, e_q)
    a_loss = COMMITMENT_COST * (2.0 * a_e_latent_loss + ae_e_latent_loss)
    e_e_latent_loss = mse(eeg_semantic, e_q)
    ea_e_latent_loss = mse(eeg_semantic, a_q)
    e_loss = COMMITMENT_COST * (2.0 * e_e_latent_loss + ea_e_latent_loss)

    a_quantized_st = audio_semantic + (a_q - audio_semantic)
    e_quantized_st = eeg_semantic + (e_q - eeg_semantic)

    a_avg_probs = jnp.sum(a_counts, axis=0) / BT
    a_perplexity = jnp.exp(-jnp.sum(a_avg_probs * jnp.log(a_avg_probs + 1e-10)))
    e_avg_probs = jnp.sum(e_counts, axis=0) / BT
    e_perplexity = jnp.exp(-jnp.sum(e_avg_probs * jnp.log(e_avg_probs + 1e-10)))

    return (a_quantized_st, e_quantized_st, a_loss, e_loss,
            a_perplexity, e_perplexity, cmcm_loss, equal_num)
